# baseline (device time: 8642 ns/iter reference)
import functools

import jax
import jax.numpy as jnp
from jax import lax
from jax.experimental import pallas as pl
from jax.experimental.pallas import tpu as pltpu


def kernel(x):
    m, n = x.shape

    def body(
        x_hbm,
        out_hbm,
        x_vmem,
        send_buf,
        recv_buf,
        out_stage,
        send_sem,
        recv_sem,
        in_sem,
        own_sem,
        oth_sem,
    ):
        my_x = lax.axis_index("x")
        my_y = lax.axis_index("y")
        my_z = lax.axis_index("z")
        other_y = 1 - my_y
        partner = (my_x, other_y, my_z)

        in_dma = pltpu.make_async_copy(x_hbm, x_vmem, in_sem)
        in_dma.start()

        barrier_sem = pltpu.get_barrier_semaphore()
        pl.semaphore_signal(
            barrier_sem, inc=1,
            device_id=partner, device_id_type=pl.DeviceIdType.MESH,
        )
        pl.semaphore_wait(barrier_sem, 1)
        in_dma.wait()

        send_buf[...] = x_vmem[...].astype(jnp.bfloat16)
        rdma = pltpu.make_async_remote_copy(
            src_ref=send_buf,
            dst_ref=recv_buf,
            send_sem=send_sem,
            recv_sem=recv_sem,
            device_id=partner,
            device_id_type=pl.DeviceIdType.MESH,
        )
        rdma.start()

        own_dma = pltpu.make_async_copy(
            x_vmem, out_hbm.at[pl.ds(my_y * m, m), :], own_sem
        )
        own_dma.start()

        rdma.wait_recv()
        out_stage[...] = recv_buf[...].astype(jnp.float32)
        oth_dma = pltpu.make_async_copy(
            out_stage, out_hbm.at[pl.ds(other_y * m, m), :], oth_sem
        )
        oth_dma.start()

        own_dma.wait()
        oth_dma.wait()
        rdma.wait_send()

        @functools.partial(
            pl.run_scoped, exit_sem=pltpu.SemaphoreType.REGULAR
        )
        def _(exit_sem):
            pl.semaphore_signal(
                exit_sem, inc=1,
                device_id=partner, device_id_type=pl.DeviceIdType.MESH,
            )
            pl.semaphore_wait(exit_sem, 1)

    return pl.pallas_call(
        body,
        out_shape=jax.ShapeDtypeStruct((2 * m, n), jnp.float32),
        in_specs=[pl.BlockSpec(memory_space=pl.ANY)],
        out_specs=pl.BlockSpec(memory_space=pl.ANY),
        scratch_shapes=[
            pltpu.VMEM((m, n), jnp.float32),
            pltpu.VMEM((m, n), jnp.bfloat16),
            pltpu.VMEM((m, n), jnp.bfloat16),
            pltpu.VMEM((m, n), jnp.float32),
            pltpu.SemaphoreType.DMA,
            pltpu.SemaphoreType.DMA,
            pltpu.SemaphoreType.DMA,
            pltpu.SemaphoreType.DMA,
            pltpu.SemaphoreType.DMA,
        ],
        compiler_params=pltpu.CompilerParams(collective_id=0),
    )(x)


# device time: 8577 ns/iter; 1.0076x vs baseline; 1.0076x over previous
import functools

import jax
import jax.numpy as jnp
from jax import lax
from jax.experimental import pallas as pl
from jax.experimental.pallas import tpu as pltpu

N_CHUNK = 2


def kernel(x):
    m, n = x.shape
    mc = m // N_CHUNK

    def body(
        x_hbm,
        out_hbm,
        x_vmem,
        send_buf,
        recv_buf,
        out_stage,
        send_sems,
        recv_sems,
        in_sem,
        own_sem,
        oth_sems,
        exit_sem,
    ):
        my_x = lax.axis_index("x")
        my_y = lax.axis_index("y")
        my_z = lax.axis_index("z")
        other_y = 1 - my_y
        partner = (my_x, other_y, my_z)

        barrier_sem = pltpu.get_barrier_semaphore()
        pl.semaphore_signal(
            barrier_sem, inc=1,
            device_id=partner, device_id_type=pl.DeviceIdType.MESH,
        )

        in_dma = pltpu.make_async_copy(x_hbm, x_vmem, in_sem)
        in_dma.start()
        in_dma.wait()
        send_buf[...] = x_vmem[...].astype(jnp.bfloat16)

        pl.semaphore_wait(barrier_sem, 1)

        rdmas = []
        for c in range(N_CHUNK):
            rows = pl.ds(c * mc, mc)
            rdma = pltpu.make_async_remote_copy(
                src_ref=send_buf.at[rows, :],
                dst_ref=recv_buf.at[rows, :],
                send_sem=send_sems.at[c],
                recv_sem=recv_sems.at[c],
                device_id=partner,
                device_id_type=pl.DeviceIdType.MESH,
            )
            rdma.start()
            rdmas.append(rdma)

        own_dma = pltpu.make_async_copy(
            x_vmem, out_hbm.at[pl.ds(my_y * m, m), :], own_sem
        )
        own_dma.start()

        oth_dmas = []
        for c in range(N_CHUNK):
            rows = pl.ds(c * mc, mc)
            rdmas[c].wait_recv()
            out_stage[rows, :] = recv_buf[rows, :].astype(jnp.float32)
            if c == N_CHUNK - 1:
                pl.semaphore_signal(
                    exit_sem, inc=1,
                    device_id=partner, device_id_type=pl.DeviceIdType.MESH,
                )
            oth_dma = pltpu.make_async_copy(
                out_stage.at[rows, :],
                out_hbm.at[pl.ds(other_y * m + c * mc, mc), :],
                oth_sems.at[c],
            )
            oth_dma.start()
            oth_dmas.append(oth_dma)

        own_dma.wait()
        for c in range(N_CHUNK):
            oth_dmas[c].wait()
            rdmas[c].wait_send()

        pl.semaphore_wait(exit_sem, 1)

    return pl.pallas_call(
        body,
        out_shape=jax.ShapeDtypeStruct((2 * m, n), jnp.float32),
        in_specs=[pl.BlockSpec(memory_space=pl.ANY)],
        out_specs=pl.BlockSpec(memory_space=pl.ANY),
        scratch_shapes=[
            pltpu.VMEM((m, n), jnp.float32),
            pltpu.VMEM((m, n), jnp.bfloat16),
            pltpu.VMEM((m, n), jnp.bfloat16),
            pltpu.VMEM((m, n), jnp.float32),
            pltpu.SemaphoreType.DMA((N_CHUNK,)),
            pltpu.SemaphoreType.DMA((N_CHUNK,)),
            pltpu.SemaphoreType.DMA,
            pltpu.SemaphoreType.DMA,
            pltpu.SemaphoreType.DMA((N_CHUNK,)),
            pltpu.SemaphoreType.REGULAR,
        ],
        compiler_params=pltpu.CompilerParams(collective_id=0),
    )(x)


# device time: 8526 ns/iter; 1.0136x vs baseline; 1.0060x over previous
import functools

import jax
import jax.numpy as jnp
from jax import lax
from jax.experimental import pallas as pl
from jax.experimental.pallas import tpu as pltpu


def kernel(x):
    m, n = x.shape

    def body(x_ref, out_ref, send_buf, recv_buf, send_sem, recv_sem):
        my_x = lax.axis_index("x")
        my_y = lax.axis_index("y")
        my_z = lax.axis_index("z")
        other_y = 1 - my_y
        partner = (my_x, other_y, my_z)

        barrier_sem = pltpu.get_barrier_semaphore()
        pl.semaphore_signal(
            barrier_sem, inc=1,
            device_id=partner, device_id_type=pl.DeviceIdType.MESH,
        )
        pl.semaphore_wait(barrier_sem, 1)

        send_buf[...] = x_ref[...].astype(jnp.bfloat16)
        rdma = pltpu.make_async_remote_copy(
            src_ref=send_buf,
            dst_ref=recv_buf,
            send_sem=send_sem,
            recv_sem=recv_sem,
            device_id=partner,
            device_id_type=pl.DeviceIdType.MESH,
        )
        rdma.start()

        out_ref[pl.ds(my_y * m, m), :] = x_ref[...]

        rdma.wait()
        out_ref[pl.ds(other_y * m, m), :] = recv_buf[...].astype(jnp.float32)

        @functools.partial(
            pl.run_scoped, exit_sem=pltpu.SemaphoreType.REGULAR
        )
        def _(exit_sem):
            pl.semaphore_signal(
                exit_sem, inc=1,
                device_id=partner, device_id_type=pl.DeviceIdType.MESH,
            )
            pl.semaphore_wait(exit_sem, 1)

    return pl.pallas_call(
        body,
        out_shape=jax.ShapeDtypeStruct((2 * m, n), jnp.float32),
        in_specs=[pl.BlockSpec(memory_space=pltpu.VMEM)],
        out_specs=pl.BlockSpec(memory_space=pltpu.VMEM),
        scratch_shapes=[
            pltpu.VMEM((m, n), jnp.bfloat16),
            pltpu.VMEM((m, n), jnp.bfloat16),
            pltpu.SemaphoreType.DMA,
            pltpu.SemaphoreType.DMA,
        ],
        compiler_params=pltpu.CompilerParams(collective_id=0),
    )(x)
